# Initial kernel scaffold; baseline (speedup 1.0000x reference)
import jax
import jax.numpy as jnp
from jax.experimental import pallas as pl


def kernel(input, incoming_links, outcoming_links, cm_W1, cm_b1, cm_W2, cm_b2, lu_W1, lu_b1, lu_W2, lu_b2, lu_W3, lu_b3, ro_W1, ro_b1, ro_W2, ro_b2, ro_W3, ro_b3):
    raise NotImplementedError("write your pallas kernel here")


# speedup vs baseline: 3.1311x; 3.1311x over previous
import functools

import jax
import jax.numpy as jnp
from jax import lax
from jax.experimental import pallas as pl
from jax.experimental.pallas import tpu as pltpu
from jax.experimental.pallas import tpu_sc as plsc

N_LINKS = 50000
E = 800000
NUM_FEATURES = 2
LS = 16
MSG_H = 64
FIRST_H = 128
FINAL_H = 64
ITERS = 4

NC, NS = 2, 16
NW = NC * NS
GCH = 128
E_PAD = 802816
LPT = 1568
N_PAD = LPT * NW
RCH = 1024
EBLK = 2048
LBLK = 2000

_mesh = plsc.VectorSubcoreMesh(core_axis_name="c", subcore_axis_name="s")
_f32 = jnp.float32


@functools.partial(
    pl.kernel,
    mesh=_mesh,
    out_type=(
        jax.ShapeDtypeStruct((E_PAD, LS), _f32),
        jax.ShapeDtypeStruct((E_PAD, LS), _f32),
    ),
    scratch_types=[
        pltpu.VMEM((GCH,), jnp.int32),
        pltpu.VMEM((GCH, LS), _f32),
        pltpu.VMEM((GCH,), jnp.int32),
        pltpu.VMEM((GCH, LS), _f32),
        pltpu.SemaphoreType.DMA,
        pltpu.SemaphoreType.DMA,
    ],
    compiler_params=pltpu.CompilerParams(use_tc_tiling_on_sc=False),
)
def _sc_gather(ls_hbm, inc_hbm, dst_hbm, oa_hbm, ob_hbm,
               idxa, rowa, idxb, rowb, sema, semb):
    wid = lax.axis_index("s") * NC + lax.axis_index("c")
    per_w = E_PAD // NW
    base = wid * per_w

    @pl.loop(0, per_w // GCH)
    def _(j):
        off = base + j * GCH
        pltpu.sync_copy(inc_hbm.at[pl.ds(off, GCH)], idxa)
        pltpu.sync_copy(dst_hbm.at[pl.ds(off, GCH)], idxb)
        ca = pltpu.async_copy(ls_hbm.at[idxa], rowa, sema)
        cb = pltpu.async_copy(ls_hbm.at[idxb], rowb, semb)
        ca.wait()
        cb.wait()
        pltpu.sync_copy(rowa, oa_hbm.at[pl.ds(off, GCH)])
        pltpu.sync_copy(rowb, ob_hbm.at[pl.ds(off, GCH)])


@functools.partial(
    pl.kernel,
    mesh=_mesh,
    out_type=(
        jax.ShapeDtypeStruct((N_PAD, LS), _f32),
        jax.ShapeDtypeStruct((N_PAD, LS), _f32),
    ),
    scratch_types=[
        pltpu.VMEM((RCH, LS), _f32),
        pltpu.VMEM((RCH,), jnp.int32),
        pltpu.VMEM((40,), jnp.int32),
        pltpu.VMEM((LPT + 8, LS), _f32),
        pltpu.VMEM((LPT + 8, LS), _f32),
    ],
    compiler_params=pltpu.CompilerParams(use_tc_tiling_on_sc=False,
                                         needs_layout_passes=False),
)
def _sc_reduce(msgs_hbm, dst_hbm, bnd_hbm, omax_hbm, omin_hbm,
               buf, dvm, bvm, accx, accn):
    i32 = jnp.int32
    wid = lax.axis_index("s") * NC + lax.axis_index("c")
    lbase = wid * LPT

    pltpu.sync_copy(bnd_hbm, bvm)

    @pl.loop(0, LPT + 8)
    def _(l):
        accx.at[l][...] = jnp.full((LS,), -jnp.inf, _f32)
        accn.at[l][...] = jnp.full((LS,), jnp.inf, _f32)

    lanes = lax.iota(i32, 16)

    def _bnd_at(k):
        c = (k // 16) * 16
        v = bvm[pl.ds(c, 16)]
        return jnp.max(jnp.where(lanes == (k - c), v, i32(-1)))

    e0 = _bnd_at(wid)
    e1 = _bnd_at(wid + 1)
    e0a = (e0 // 16) * 16
    e1a = ((e1 + 15) // 16) * 16
    nch = lax.div(e1a - e0a + (RCH - 1), RCH)

    @pl.loop(0, nch)
    def _(j):
        p = e0a + j * RCH
        pltpu.sync_copy(msgs_hbm.at[pl.ds(p, RCH)], buf)
        pltpu.sync_copy(dst_hbm.at[pl.ds(p, RCH)], dvm)
        nv = lax.div(jnp.minimum(i32(RCH), e1a - p), i32(16))

        @pl.loop(0, nv)
        def _(q):
            dvec = dvm[pl.ds(q * 16, 16)] - lbase

            for i in range(16):
                loc = jnp.max(jnp.where(lanes == i, dvec, i32(-1)))
                loc = jnp.where((loc >= 0) & (loc < LPT), loc, i32(LPT))
                row = buf.at[q * 16 + i][...]
                accx.at[loc][...] = jnp.maximum(accx.at[loc][...], row)
                accn.at[loc][...] = jnp.minimum(accn.at[loc][...], row)

    pltpu.sync_copy(accx.at[pl.ds(0, LPT)], omax_hbm.at[pl.ds(lbase, LPT)])
    pltpu.sync_copy(accn.at[pl.ds(0, LPT)], omin_hbm.at[pl.ds(lbase, LPT)])


def _edge_mlp_body(xa_ref, xb_ref, wa_ref, wb_ref, b1_ref, w2_ref, b2_ref,
                   o_ref):
    h = jnp.tanh(
        jnp.dot(xa_ref[...], wa_ref[...], preferred_element_type=_f32)
        + jnp.dot(xb_ref[...], wb_ref[...], preferred_element_type=_f32)
        + b1_ref[...])
    o_ref[...] = jnp.tanh(
        jnp.dot(h, w2_ref[...], preferred_element_type=_f32) + b2_ref[...])


def _edge_mlp(inc_g, outc_g, wa, wb, b1, w2, b2):
    grid = (E_PAD // EBLK,)
    return pl.pallas_call(
        _edge_mlp_body,
        grid=grid,
        in_specs=[
            pl.BlockSpec((EBLK, LS), lambda i: (i, 0)),
            pl.BlockSpec((EBLK, LS), lambda i: (i, 0)),
            pl.BlockSpec((LS, MSG_H), lambda i: (0, 0)),
            pl.BlockSpec((LS, MSG_H), lambda i: (0, 0)),
            pl.BlockSpec((1, MSG_H), lambda i: (0, 0)),
            pl.BlockSpec((MSG_H, LS), lambda i: (0, 0)),
            pl.BlockSpec((1, LS), lambda i: (0, 0)),
        ],
        out_specs=pl.BlockSpec((EBLK, LS), lambda i: (i, 0)),
        out_shape=jax.ShapeDtypeStruct((E_PAD, LS), _f32),
    )(inc_g, outc_g, wa, wb, b1, w2, b2)


def _link_mlp_body(ls_ref, ax_ref, an_ref, wa_ref, wb_ref, wc_ref, b1_ref,
                   w2_ref, b2_ref, w3_ref, b3_ref, o_ref):
    h1 = jnp.tanh(
        jnp.dot(ls_ref[...], wa_ref[...], preferred_element_type=_f32)
        + jnp.dot(ax_ref[...], wb_ref[...], preferred_element_type=_f32)
        + jnp.dot(an_ref[...], wc_ref[...], preferred_element_type=_f32)
        + b1_ref[...])
    h2 = jnp.tanh(
        jnp.dot(h1, w2_ref[...], preferred_element_type=_f32) + b2_ref[...])
    o_ref[...] = jnp.tanh(
        jnp.dot(h2, w3_ref[...], preferred_element_type=_f32) + b3_ref[...])


def _link_mlp(ls, amax, amin, wa, wb, wc, b1, w2, b2, w3, b3):
    grid = (N_LINKS // LBLK,)
    return pl.pallas_call(
        _link_mlp_body,
        grid=grid,
        in_specs=[
            pl.BlockSpec((LBLK, LS), lambda i: (i, 0)),
            pl.BlockSpec((LBLK, LS), lambda i: (i, 0)),
            pl.BlockSpec((LBLK, LS), lambda i: (i, 0)),
            pl.BlockSpec((LS, FIRST_H), lambda i: (0, 0)),
            pl.BlockSpec((LS, FIRST_H), lambda i: (0, 0)),
            pl.BlockSpec((LS, FIRST_H), lambda i: (0, 0)),
            pl.BlockSpec((1, FIRST_H), lambda i: (0, 0)),
            pl.BlockSpec((FIRST_H, FINAL_H), lambda i: (0, 0)),
            pl.BlockSpec((1, FINAL_H), lambda i: (0, 0)),
            pl.BlockSpec((FINAL_H, LS), lambda i: (0, 0)),
            pl.BlockSpec((1, LS), lambda i: (0, 0)),
        ],
        out_specs=pl.BlockSpec((LBLK, LS), lambda i: (i, 0)),
        out_shape=jax.ShapeDtypeStruct((N_LINKS, LS), _f32),
    )(ls, amax, amin, wa, wb, wc, b1, w2, b2, w3, b3)


def _readout_body(ls_ref, w1_ref, b1_ref, w2_ref, b2_ref, w3_ref, b3_ref,
                  o_ref, s_acc, q_acc, x_acc, n_acc):
    i = pl.program_id(0)
    blk = ls_ref[...]
    s = jnp.sum(blk, axis=0, keepdims=True)
    q = jnp.sum(blk * blk, axis=0, keepdims=True)
    x = jnp.max(blk, axis=0, keepdims=True)
    n = jnp.min(blk, axis=0, keepdims=True)

    @pl.when(i == 0)
    def _():
        s_acc[...] = s
        q_acc[...] = q
        x_acc[...] = x
        n_acc[...] = n

    @pl.when(i > 0)
    def _():
        s_acc[...] = s_acc[...] + s
        q_acc[...] = q_acc[...] + q
        x_acc[...] = jnp.maximum(x_acc[...], x)
        n_acc[...] = jnp.minimum(n_acc[...], n)

    @pl.when(i == (N_LINKS // LBLK) - 1)
    def _():
        inv = _f32(1.0 / N_LINKS)
        mean = s_acc[...] * inv
        var = q_acc[...] * inv - mean * mean
        std = jnp.sqrt(jnp.maximum(var, 0.0))
        ri = jnp.concatenate([mean, x_acc[...], n_acc[...], std], axis=1)
        r1 = jnp.tanh(
            jnp.dot(ri, w1_ref[...], preferred_element_type=_f32) + b1_ref[...])
        r2 = jnp.tanh(
            jnp.dot(r1, w2_ref[...], preferred_element_type=_f32) + b2_ref[...])
        o_ref[...] = (
            jnp.dot(r2, w3_ref[...], preferred_element_type=_f32) + b3_ref[...])


def _readout(ls, w1, b1, w2, b2, w3, b3):
    grid = (N_LINKS // LBLK,)
    return pl.pallas_call(
        _readout_body,
        grid=grid,
        in_specs=[
            pl.BlockSpec((LBLK, LS), lambda i: (i, 0)),
            pl.BlockSpec((4 * LS, FIRST_H), lambda i: (0, 0)),
            pl.BlockSpec((1, FIRST_H), lambda i: (0, 0)),
            pl.BlockSpec((FIRST_H, FINAL_H), lambda i: (0, 0)),
            pl.BlockSpec((1, FINAL_H), lambda i: (0, 0)),
            pl.BlockSpec((FINAL_H, 1), lambda i: (0, 0)),
            pl.BlockSpec((1, 1), lambda i: (0, 0)),
        ],
        out_specs=pl.BlockSpec((1, 1), lambda i: (0, 0)),
        out_shape=jax.ShapeDtypeStruct((1, 1), _f32),
        scratch_shapes=[
            pltpu.VMEM((1, LS), _f32),
            pltpu.VMEM((1, LS), _f32),
            pltpu.VMEM((1, LS), _f32),
            pltpu.VMEM((1, LS), _f32),
        ],
    )(ls, w1, b1, w2, b2, w3, b3)


def kernel(input, incoming_links, outcoming_links,
           cm_W1, cm_b1, cm_W2, cm_b2,
           lu_W1, lu_b1, lu_W2, lu_b2, lu_W3, lu_b3,
           ro_W1, ro_b1, ro_W2, ro_b2, ro_W3, ro_b3):
    i32 = jnp.int32
    ls = jnp.pad(jnp.transpose(input.reshape(NUM_FEATURES, N_LINKS)),
                 ((0, 0), (0, LS - NUM_FEATURES)))

    dst = outcoming_links.astype(i32)
    inc = incoming_links.astype(i32)
    dst_s, inc_s = lax.sort_key_val(dst, inc)
    bnd = jnp.searchsorted(dst_s, jnp.arange(NW + 1, dtype=i32) * LPT,
                           ).astype(i32)
    bnd = jnp.pad(bnd, (0, 40 - (NW + 1)), constant_values=E)
    pad_e = jnp.zeros((E_PAD - E,), i32)
    dst_p = jnp.concatenate([dst_s, pad_e])
    inc_p = jnp.concatenate([inc_s, pad_e])

    cm_wa, cm_wb = cm_W1[:LS], cm_W1[LS:]
    lu_wa, lu_wb, lu_wc = lu_W1[:LS], lu_W1[LS:2 * LS], lu_W1[2 * LS:]
    cb1 = cm_b1.reshape(1, -1)
    cb2 = cm_b2.reshape(1, -1)
    lb1 = lu_b1.reshape(1, -1)
    lb2 = lu_b2.reshape(1, -1)
    lb3 = lu_b3.reshape(1, -1)
    rb1 = ro_b1.reshape(1, -1)
    rb2 = ro_b2.reshape(1, -1)
    rb3 = ro_b3.reshape(1, -1)

    for _ in range(ITERS):
        inc_g, outc_g = _sc_gather(ls, inc_p, dst_p)
        msgs = _edge_mlp(inc_g, outc_g, cm_wa, cm_wb, cb1, cm_W2, cb2)
        amax_p, amin_p = _sc_reduce(msgs, dst_p, bnd)
        ls = _link_mlp(ls, amax_p[:N_LINKS], amin_p[:N_LINKS],
                       lu_wa, lu_wb, lu_wc, lb1, lu_W2, lb2, lu_W3, lb3)

    v = _readout(ls, ro_W1, rb1, ro_W2, rb2, ro_W3, rb3)
    return v.reshape(-1)


# speedup vs baseline: 6.2498x; 1.9960x over previous
import functools

import jax
import jax.numpy as jnp
from jax import lax
from jax.experimental import pallas as pl
from jax.experimental.pallas import tpu as pltpu
from jax.experimental.pallas import tpu_sc as plsc

N_LINKS = 50000
E = 800000
NUM_FEATURES = 2
LS = 16
MSG_H = 64
FIRST_H = 128
FINAL_H = 64
ITERS = 4

NC, NS = 2, 16
NW = NC * NS
GCH = 128
E_PAD = 802816
E8 = E_PAD // 8
LPT = 1568
N_PAD = LPT * NW
N8 = N_PAD // 8
R8 = N_LINKS * LS // 128
RCH = 1024
EBLK8 = 512
LBLK8 = 784

_mesh = plsc.VectorSubcoreMesh(core_axis_name="c", subcore_axis_name="s")
_f32 = jnp.float32


@functools.partial(
    pl.kernel,
    mesh=_mesh,
    out_type=(
        jax.ShapeDtypeStruct((E_PAD, LS), _f32),
        jax.ShapeDtypeStruct((E_PAD, LS), _f32),
    ),
    scratch_types=[
        pltpu.VMEM((GCH,), jnp.int32),
        pltpu.VMEM((GCH, LS), _f32),
        pltpu.VMEM((GCH,), jnp.int32),
        pltpu.VMEM((GCH, LS), _f32),
        pltpu.SemaphoreType.DMA,
        pltpu.SemaphoreType.DMA,
    ],
    compiler_params=pltpu.CompilerParams(use_tc_tiling_on_sc=False),
)
def _sc_gather(ls_hbm, inc_hbm, dst_hbm, oa_hbm, ob_hbm,
               idxa, rowa, idxb, rowb, sema, semb):
    wid = lax.axis_index("s") * NC + lax.axis_index("c")
    per_w = E_PAD // NW
    base = wid * per_w

    @pl.loop(0, per_w // GCH)
    def _(j):
        off = base + j * GCH
        pltpu.sync_copy(inc_hbm.at[pl.ds(off, GCH)], idxa)
        pltpu.sync_copy(dst_hbm.at[pl.ds(off, GCH)], idxb)
        ca = pltpu.async_copy(ls_hbm.at[idxa], rowa, sema)
        cb = pltpu.async_copy(ls_hbm.at[idxb], rowb, semb)
        ca.wait()
        cb.wait()
        pltpu.sync_copy(rowa, oa_hbm.at[pl.ds(off, GCH)])
        pltpu.sync_copy(rowb, ob_hbm.at[pl.ds(off, GCH)])


@functools.partial(
    pl.kernel,
    mesh=_mesh,
    out_type=(
        jax.ShapeDtypeStruct((N_PAD, LS), _f32),
        jax.ShapeDtypeStruct((N_PAD, LS), _f32),
    ),
    scratch_types=[
        pltpu.VMEM((RCH, LS), _f32),
        pltpu.VMEM((RCH,), jnp.int32),
        pltpu.VMEM((40,), jnp.int32),
        pltpu.VMEM((LPT + 8, LS), _f32),
        pltpu.VMEM((LPT + 8, LS), _f32),
    ],
    compiler_params=pltpu.CompilerParams(use_tc_tiling_on_sc=False,
                                         needs_layout_passes=False),
)
def _sc_reduce(msgs_hbm, dst_hbm, bnd_hbm, omax_hbm, omin_hbm,
               buf, dvm, bvm, accx, accn):
    i32 = jnp.int32
    wid = lax.axis_index("s") * NC + lax.axis_index("c")
    lbase = wid * LPT

    pltpu.sync_copy(bnd_hbm, bvm)

    @pl.loop(0, LPT + 8)
    def _(l):
        accx.at[l][...] = jnp.full((LS,), -jnp.inf, _f32)
        accn.at[l][...] = jnp.full((LS,), jnp.inf, _f32)

    lanes = lax.iota(i32, 16)

    def _bnd_at(k):
        c = (k // 16) * 16
        v = bvm[pl.ds(c, 16)]
        return jnp.max(jnp.where(lanes == (k - c), v, i32(-1)))

    e0 = _bnd_at(wid)
    e1 = _bnd_at(wid + 1)
    e0a = (e0 // 16) * 16
    e1a = ((e1 + 15) // 16) * 16
    nch = lax.div(e1a - e0a + (RCH - 1), RCH)

    @pl.loop(0, nch)
    def _(j):
        p = e0a + j * RCH
        pltpu.sync_copy(msgs_hbm.at[pl.ds(p, RCH)], buf)
        pltpu.sync_copy(dst_hbm.at[pl.ds(p, RCH)], dvm)
        nv = lax.div(jnp.minimum(i32(RCH), e1a - p), i32(16))

        @pl.loop(0, nv)
        def _(q):
            dvec = dvm[pl.ds(q * 16, 16)] - lbase

            for i in range(16):
                loc = jnp.max(jnp.where(lanes == i, dvec, i32(-1)))
                loc = jnp.where((loc >= 0) & (loc < LPT), loc, i32(LPT))
                row = buf.at[q * 16 + i][...]
                accx.at[loc][...] = jnp.maximum(accx.at[loc][...], row)
                accn.at[loc][...] = jnp.minimum(accn.at[loc][...], row)

    pltpu.sync_copy(accx.at[pl.ds(0, LPT)], omax_hbm.at[pl.ds(lbase, LPT)])
    pltpu.sync_copy(accn.at[pl.ds(0, LPT)], omin_hbm.at[pl.ds(lbase, LPT)])


def _edge_mlp_body(xa_ref, xb_ref, wa_ref, wb_ref, b1_ref, w2_ref, b2_ref,
                   o_ref):
    h = jnp.tanh(
        jnp.dot(xa_ref[...], wa_ref[...], preferred_element_type=_f32)
        + jnp.dot(xb_ref[...], wb_ref[...], preferred_element_type=_f32)
        + b1_ref[...])
    o_ref[...] = jnp.tanh(
        jnp.dot(h, w2_ref[...], preferred_element_type=_f32) + b2_ref[...])


def _edge_mlp(xa, xb, wa, wb, b1, w2, b2):
    grid = (E8 // EBLK8,)
    return pl.pallas_call(
        _edge_mlp_body,
        grid=grid,
        in_specs=[
            pl.BlockSpec((EBLK8, 128), lambda i: (i, 0)),
            pl.BlockSpec((EBLK8, 128), lambda i: (i, 0)),
            pl.BlockSpec((128, 8 * MSG_H), lambda i: (0, 0)),
            pl.BlockSpec((128, 8 * MSG_H), lambda i: (0, 0)),
            pl.BlockSpec((1, 8 * MSG_H), lambda i: (0, 0)),
            pl.BlockSpec((8 * MSG_H, 128), lambda i: (0, 0)),
            pl.BlockSpec((1, 128), lambda i: (0, 0)),
        ],
        out_specs=pl.BlockSpec((EBLK8, 128), lambda i: (i, 0)),
        out_shape=jax.ShapeDtypeStruct((E8, 128), _f32),
    )(xa, xb, wa, wb, b1, w2, b2)


def _link_mlp_body(ls_ref, ax_ref, an_ref, wa_ref, wb_ref, wc_ref, b1_ref,
                   w2_ref, b2_ref, w3_ref, b3_ref, o_ref):
    h1 = jnp.tanh(
        jnp.dot(ls_ref[...], wa_ref[...], preferred_element_type=_f32)
        + jnp.dot(ax_ref[...], wb_ref[...], preferred_element_type=_f32)
        + jnp.dot(an_ref[...], wc_ref[...], preferred_element_type=_f32)
        + b1_ref[...])
    h2 = jnp.tanh(
        jnp.dot(h1, w2_ref[...], preferred_element_type=_f32) + b2_ref[...])
    o_ref[...] = jnp.tanh(
        jnp.dot(h2, w3_ref[...], preferred_element_type=_f32) + b3_ref[...])


def _link_mlp(ls, amax, amin, wa, wb, wc, b1, w2, b2, w3, b3):
    grid = (N8 // LBLK8,)
    return pl.pallas_call(
        _link_mlp_body,
        grid=grid,
        in_specs=[
            pl.BlockSpec((LBLK8, 128), lambda i: (i, 0)),
            pl.BlockSpec((LBLK8, 128), lambda i: (i, 0)),
            pl.BlockSpec((LBLK8, 128), lambda i: (i, 0)),
            pl.BlockSpec((128, 8 * FIRST_H), lambda i: (0, 0)),
            pl.BlockSpec((128, 8 * FIRST_H), lambda i: (0, 0)),
            pl.BlockSpec((128, 8 * FIRST_H), lambda i: (0, 0)),
            pl.BlockSpec((1, 8 * FIRST_H), lambda i: (0, 0)),
            pl.BlockSpec((8 * FIRST_H, 8 * FINAL_H), lambda i: (0, 0)),
            pl.BlockSpec((1, 8 * FINAL_H), lambda i: (0, 0)),
            pl.BlockSpec((8 * FINAL_H, 128), lambda i: (0, 0)),
            pl.BlockSpec((1, 128), lambda i: (0, 0)),
        ],
        out_specs=pl.BlockSpec((LBLK8, 128), lambda i: (i, 0)),
        out_shape=jax.ShapeDtypeStruct((N8, 128), _f32),
    )(ls, amax, amin, wa, wb, wc, b1, w2, b2, w3, b3)


def _readout_body(ls_ref, w1_ref, b1_ref, w2_ref, b2_ref, w3_ref, b3_ref,
                  o_ref, s_acc, q_acc, x_acc, n_acc):
    i = pl.program_id(0)
    rows = lax.broadcasted_iota(jnp.int32, (LBLK8, 128), 0) + i * LBLK8
    valid = rows < R8
    blk = ls_ref[...]
    zero = jnp.zeros_like(blk)
    bs = jnp.where(valid, blk, zero)
    s = jnp.sum(bs, axis=0, keepdims=True)
    q = jnp.sum(bs * bs, axis=0, keepdims=True)
    x = jnp.max(jnp.where(valid, blk, -jnp.inf), axis=0, keepdims=True)
    n = jnp.min(jnp.where(valid, blk, jnp.inf), axis=0, keepdims=True)

    @pl.when(i == 0)
    def _():
        s_acc[...] = s
        q_acc[...] = q
        x_acc[...] = x
        n_acc[...] = n

    @pl.when(i > 0)
    def _():
        s_acc[...] = s_acc[...] + s
        q_acc[...] = q_acc[...] + q
        x_acc[...] = jnp.maximum(x_acc[...], x)
        n_acc[...] = jnp.minimum(n_acc[...], n)

    @pl.when(i == (N8 // LBLK8) - 1)
    def _():
        def _fold(v, op):
            parts = [v[:, k * LS:(k + 1) * LS] for k in range(8)]
            r = parts[0]
            for p in parts[1:]:
                r = op(r, p)
            return r

        s16 = _fold(s_acc[...], jnp.add)
        q16 = _fold(q_acc[...], jnp.add)
        x16 = _fold(x_acc[...], jnp.maximum)
        n16 = _fold(n_acc[...], jnp.minimum)
        inv = _f32(1.0 / N_LINKS)
        mean = s16 * inv
        var = q16 * inv - mean * mean
        std = jnp.sqrt(jnp.maximum(var, 0.0))
        ri = jnp.concatenate([mean, x16, n16, std], axis=1)
        r1 = jnp.tanh(
            jnp.dot(ri, w1_ref[...], preferred_element_type=_f32) + b1_ref[...])
        r2 = jnp.tanh(
            jnp.dot(r1, w2_ref[...], preferred_element_type=_f32) + b2_ref[...])
        o_ref[...] = (
            jnp.dot(r2, w3_ref[...], preferred_element_type=_f32) + b3_ref[...])


def _readout(ls, w1, b1, w2, b2, w3, b3):
    grid = (N8 // LBLK8,)
    return pl.pallas_call(
        _readout_body,
        grid=grid,
        in_specs=[
            pl.BlockSpec((LBLK8, 128), lambda i: (i, 0)),
            pl.BlockSpec((4 * LS, FIRST_H), lambda i: (0, 0)),
            pl.BlockSpec((1, FIRST_H), lambda i: (0, 0)),
            pl.BlockSpec((FIRST_H, FINAL_H), lambda i: (0, 0)),
            pl.BlockSpec((1, FINAL_H), lambda i: (0, 0)),
            pl.BlockSpec((FINAL_H, 1), lambda i: (0, 0)),
            pl.BlockSpec((1, 1), lambda i: (0, 0)),
        ],
        out_specs=pl.BlockSpec((1, 1), lambda i: (0, 0)),
        out_shape=jax.ShapeDtypeStruct((1, 1), _f32),
        scratch_shapes=[
            pltpu.VMEM((1, 128), _f32),
            pltpu.VMEM((1, 128), _f32),
            pltpu.VMEM((1, 128), _f32),
            pltpu.VMEM((1, 128), _f32),
        ],
    )(ls, w1, b1, w2, b2, w3, b3)


def _bd(w):
    return jnp.kron(jnp.eye(8, dtype=_f32), w)


def kernel(input, incoming_links, outcoming_links,
           cm_W1, cm_b1, cm_W2, cm_b2,
           lu_W1, lu_b1, lu_W2, lu_b2, lu_W3, lu_b3,
           ro_W1, ro_b1, ro_W2, ro_b2, ro_W3, ro_b3):
    i32 = jnp.int32
    ls0 = jnp.pad(jnp.transpose(input.reshape(NUM_FEATURES, N_LINKS)),
                  ((0, N_PAD - N_LINKS), (0, LS - NUM_FEATURES)))
    ls = ls0.reshape(N8, 128)

    dst = outcoming_links.astype(i32)
    inc = incoming_links.astype(i32)
    dst_s, inc_s = lax.sort_key_val(dst, inc)
    bnd = jnp.searchsorted(dst_s, jnp.arange(NW + 1, dtype=i32) * LPT,
                           ).astype(i32)
    bnd = jnp.pad(bnd, (0, 40 - (NW + 1)), constant_values=E)
    pad_e = jnp.zeros((E_PAD - E,), i32)
    dst_p = jnp.concatenate([dst_s, pad_e])
    inc_p = jnp.concatenate([inc_s, pad_e])

    cm_wa, cm_wb = _bd(cm_W1[:LS]), _bd(cm_W1[LS:])
    cm_w2 = _bd(cm_W2)
    lu_wa, lu_wb, lu_wc = (_bd(lu_W1[:LS]), _bd(lu_W1[LS:2 * LS]),
                           _bd(lu_W1[2 * LS:]))
    lu_w2, lu_w3 = _bd(lu_W2), _bd(lu_W3)
    cb1 = jnp.tile(cm_b1, 8).reshape(1, -1)
    cb2 = jnp.tile(cm_b2, 8).reshape(1, -1)
    lb1 = jnp.tile(lu_b1, 8).reshape(1, -1)
    lb2 = jnp.tile(lu_b2, 8).reshape(1, -1)
    lb3 = jnp.tile(lu_b3, 8).reshape(1, -1)
    rb1 = ro_b1.reshape(1, -1)
    rb2 = ro_b2.reshape(1, -1)
    rb3 = ro_b3.reshape(1, -1)

    for _ in range(ITERS):
        inc_g, outc_g = _sc_gather(ls.reshape(N_PAD, LS), inc_p, dst_p)
        msgs = _edge_mlp(inc_g.reshape(E8, 128), outc_g.reshape(E8, 128),
                         cm_wa, cm_wb, cb1, cm_w2, cb2)
        amax_p, amin_p = _sc_reduce(msgs.reshape(E_PAD, LS), dst_p, bnd)
        ls = _link_mlp(ls, amax_p.reshape(N8, 128), amin_p.reshape(N8, 128),
                       lu_wa, lu_wb, lu_wc, lb1, lu_w2, lb2, lu_w3, lb3)

    v = _readout(ls, ro_W1, rb1, ro_W2, rb2, ro_W3, rb3)
    return v.reshape(-1)


# speedup vs baseline: 7.5697x; 1.2112x over previous
import functools

import jax
import jax.numpy as jnp
from jax import lax
from jax.experimental import pallas as pl
from jax.experimental.pallas import tpu as pltpu
from jax.experimental.pallas import tpu_sc as plsc

N_LINKS = 50000
E = 800000
NUM_FEATURES = 2
LS = 16
MSG_H = 64
FIRST_H = 128
FINAL_H = 64
ITERS = 4

NC, NS = 2, 16
NW = NC * NS
GCH = 128
E_PAD = 802816
E8 = E_PAD // 8
LPT = 1568
N_PAD = LPT * NW
N8 = N_PAD // 8
R8 = N_LINKS * LS // 128
RCH = 1024
EBLK8 = 512
LBLK8 = 784

_mesh = plsc.VectorSubcoreMesh(core_axis_name="c", subcore_axis_name="s")
_f32 = jnp.float32


GROWS = GCH
NGRP = (E_PAD // NW) // GROWS
NCHT = E_PAD // NW // GCH


@functools.partial(
    pl.kernel,
    mesh=_mesh,
    out_type=(
        jax.ShapeDtypeStruct((E_PAD, LS), _f32),
        jax.ShapeDtypeStruct((E_PAD, LS), _f32),
    ),
    scratch_types=[
        pltpu.VMEM((NCHT, GCH), jnp.int32),
        pltpu.VMEM((NCHT, GCH), jnp.int32),
        pltpu.VMEM((GROWS, LS), _f32),
        pltpu.VMEM((GROWS, LS), _f32),
        pltpu.VMEM((GROWS, LS), _f32),
        pltpu.VMEM((GROWS, LS), _f32),
        pltpu.SemaphoreType.DMA,
        pltpu.SemaphoreType.DMA,
        pltpu.SemaphoreType.DMA,
        pltpu.SemaphoreType.DMA,
    ],
    compiler_params=pltpu.CompilerParams(use_tc_tiling_on_sc=False),
)
def _sc_gather(ls_hbm, inc_hbm, dst_hbm, oa_hbm, ob_hbm,
               ivi, ivd, ra0, ra1, rb0, rb1, sg0, sg1, so0, so1):
    wid = lax.axis_index("s") * NC + lax.axis_index("c")
    base = wid * (E_PAD // NW)
    ras, rbs = (ra0, ra1), (rb0, rb1)
    sgs, sos = (sg0, sg1), (so0, so1)

    pltpu.sync_copy(inc_hbm.at[pl.ds(wid * NCHT, NCHT)], ivi)
    pltpu.sync_copy(dst_hbm.at[pl.ds(wid * NCHT, NCHT)], ivd)

    def fire_g(g, s):
        pltpu.async_copy(ls_hbm.at[ivi.at[g]], ras[s], sgs[s])
        pltpu.async_copy(ls_hbm.at[ivd.at[g]], rbs[s], sgs[s])

    def wait2(sem, s):
        pltpu.make_async_copy(oa_hbm.at[pl.ds(0, GROWS)], ras[s], sem).wait()
        pltpu.make_async_copy(oa_hbm.at[pl.ds(0, GROWS)], rbs[s], sem).wait()

    def fire_o(g, s):
        off = base + g * GROWS
        pltpu.async_copy(ras[s], oa_hbm.at[pl.ds(off, GROWS)], sos[s])
        pltpu.async_copy(rbs[s], ob_hbm.at[pl.ds(off, GROWS)], sos[s])

    fire_g(0, 0)

    @pl.loop(0, NGRP // 2)
    def _(t):
        for u in range(2):
            g = 2 * t + u
            wait2(sgs[u], u)
            fire_o(g, u)

            @pl.when(g + 1 < NGRP)
            def _():
                @pl.when(g >= 1)
                def _():
                    wait2(sos[1 - u], 1 - u)

                fire_g(g + 1, 1 - u)

    wait2(sos[0], 0)
    wait2(sos[1], 1)


@functools.partial(
    pl.kernel,
    mesh=_mesh,
    out_type=(
        jax.ShapeDtypeStruct((N_PAD, LS), _f32),
        jax.ShapeDtypeStruct((N_PAD, LS), _f32),
    ),
    scratch_types=[
        pltpu.VMEM((RCH, LS), _f32),
        pltpu.VMEM((RCH,), jnp.int32),
        pltpu.VMEM((40,), jnp.int32),
        pltpu.VMEM((LPT + 8, LS), _f32),
        pltpu.VMEM((LPT + 8, LS), _f32),
    ],
    compiler_params=pltpu.CompilerParams(use_tc_tiling_on_sc=False,
                                         needs_layout_passes=False),
)
def _sc_reduce(msgs_hbm, dst_hbm, bnd_hbm, omax_hbm, omin_hbm,
               buf, dvm, bvm, accx, accn):
    i32 = jnp.int32
    wid = lax.axis_index("s") * NC + lax.axis_index("c")
    lbase = wid * LPT

    pltpu.sync_copy(bnd_hbm, bvm)

    @pl.loop(0, LPT + 8)
    def _(l):
        accx.at[l][...] = jnp.full((LS,), -jnp.inf, _f32)
        accn.at[l][...] = jnp.full((LS,), jnp.inf, _f32)

    lanes = lax.iota(i32, 16)

    def _bnd_at(k):
        c = (k // 16) * 16
        v = bvm[pl.ds(c, 16)]
        return jnp.max(jnp.where(lanes == (k - c), v, i32(-1)))

    e0 = _bnd_at(wid)
    e1 = _bnd_at(wid + 1)
    e0a = (e0 // 16) * 16
    e1a = ((e1 + 15) // 16) * 16
    nch = lax.div(e1a - e0a + (RCH - 1), RCH)

    @pl.loop(0, nch)
    def _(j):
        p = e0a + j * RCH
        pltpu.sync_copy(msgs_hbm.at[pl.ds(p, RCH)], buf)
        pltpu.sync_copy(dst_hbm.at[pl.ds(p, RCH)], dvm)
        nv = lax.div(jnp.minimum(i32(RCH), e1a - p), i32(16))

        @pl.loop(0, nv)
        def _(q):
            dvec = dvm[pl.ds(q * 16, 16)] - lbase

            for i in range(16):
                loc = jnp.max(jnp.where(lanes == i, dvec, i32(-1)))
                loc = jnp.where((loc >= 0) & (loc < LPT), loc, i32(LPT))
                row = buf.at[q * 16 + i][...]
                accx.at[loc][...] = jnp.maximum(accx.at[loc][...], row)
                accn.at[loc][...] = jnp.minimum(accn.at[loc][...], row)

    pltpu.sync_copy(accx.at[pl.ds(0, LPT)], omax_hbm.at[pl.ds(lbase, LPT)])
    pltpu.sync_copy(accn.at[pl.ds(0, LPT)], omin_hbm.at[pl.ds(lbase, LPT)])


def _edge_mlp_body(xa_ref, xb_ref, wa_ref, wb_ref, b1_ref, w2_ref, b2_ref,
                   o_ref):
    h = jnp.tanh(
        jnp.dot(xa_ref[...], wa_ref[...], preferred_element_type=_f32)
        + jnp.dot(xb_ref[...], wb_ref[...], preferred_element_type=_f32)
        + b1_ref[...])
    o_ref[...] = jnp.tanh(
        jnp.dot(h, w2_ref[...], preferred_element_type=_f32) + b2_ref[...])


def _edge_mlp(xa, xb, wa, wb, b1, w2, b2):
    grid = (E8 // EBLK8,)
    return pl.pallas_call(
        _edge_mlp_body,
        grid=grid,
        in_specs=[
            pl.BlockSpec((EBLK8, 128), lambda i: (i, 0)),
            pl.BlockSpec((EBLK8, 128), lambda i: (i, 0)),
            pl.BlockSpec((128, 8 * MSG_H), lambda i: (0, 0)),
            pl.BlockSpec((128, 8 * MSG_H), lambda i: (0, 0)),
            pl.BlockSpec((1, 8 * MSG_H), lambda i: (0, 0)),
            pl.BlockSpec((8 * MSG_H, 128), lambda i: (0, 0)),
            pl.BlockSpec((1, 128), lambda i: (0, 0)),
        ],
        out_specs=pl.BlockSpec((EBLK8, 128), lambda i: (i, 0)),
        out_shape=jax.ShapeDtypeStruct((E8, 128), _f32),
    )(xa, xb, wa, wb, b1, w2, b2)


def _link_mlp_body(ls_ref, ax_ref, an_ref, wa_ref, wb_ref, wc_ref, b1_ref,
                   w2_ref, b2_ref, w3_ref, b3_ref, o_ref):
    h1 = jnp.tanh(
        jnp.dot(ls_ref[...], wa_ref[...], preferred_element_type=_f32)
        + jnp.dot(ax_ref[...], wb_ref[...], preferred_element_type=_f32)
        + jnp.dot(an_ref[...], wc_ref[...], preferred_element_type=_f32)
        + b1_ref[...])
    h2 = jnp.tanh(
        jnp.dot(h1, w2_ref[...], preferred_element_type=_f32) + b2_ref[...])
    o_ref[...] = jnp.tanh(
        jnp.dot(h2, w3_ref[...], preferred_element_type=_f32) + b3_ref[...])


def _link_mlp(ls, amax, amin, wa, wb, wc, b1, w2, b2, w3, b3):
    grid = (N8 // LBLK8,)
    return pl.pallas_call(
        _link_mlp_body,
        grid=grid,
        in_specs=[
            pl.BlockSpec((LBLK8, 128), lambda i: (i, 0)),
            pl.BlockSpec((LBLK8, 128), lambda i: (i, 0)),
            pl.BlockSpec((LBLK8, 128), lambda i: (i, 0)),
            pl.BlockSpec((128, 8 * FIRST_H), lambda i: (0, 0)),
            pl.BlockSpec((128, 8 * FIRST_H), lambda i: (0, 0)),
            pl.BlockSpec((128, 8 * FIRST_H), lambda i: (0, 0)),
            pl.BlockSpec((1, 8 * FIRST_H), lambda i: (0, 0)),
            pl.BlockSpec((8 * FIRST_H, 8 * FINAL_H), lambda i: (0, 0)),
            pl.BlockSpec((1, 8 * FINAL_H), lambda i: (0, 0)),
            pl.BlockSpec((8 * FINAL_H, 128), lambda i: (0, 0)),
            pl.BlockSpec((1, 128), lambda i: (0, 0)),
        ],
        out_specs=pl.BlockSpec((LBLK8, 128), lambda i: (i, 0)),
        out_shape=jax.ShapeDtypeStruct((N8, 128), _f32),
    )(ls, amax, amin, wa, wb, wc, b1, w2, b2, w3, b3)


def _readout_body(ls_ref, w1_ref, b1_ref, w2_ref, b2_ref, w3_ref, b3_ref,
                  o_ref, s_acc, q_acc, x_acc, n_acc):
    i = pl.program_id(0)
    rows = lax.broadcasted_iota(jnp.int32, (LBLK8, 128), 0) + i * LBLK8
    valid = rows < R8
    blk = ls_ref[...]
    zero = jnp.zeros_like(blk)
    bs = jnp.where(valid, blk, zero)
    s = jnp.sum(bs, axis=0, keepdims=True)
    q = jnp.sum(bs * bs, axis=0, keepdims=True)
    x = jnp.max(jnp.where(valid, blk, -jnp.inf), axis=0, keepdims=True)
    n = jnp.min(jnp.where(valid, blk, jnp.inf), axis=0, keepdims=True)

    @pl.when(i == 0)
    def _():
        s_acc[...] = s
        q_acc[...] = q
        x_acc[...] = x
        n_acc[...] = n

    @pl.when(i > 0)
    def _():
        s_acc[...] = s_acc[...] + s
        q_acc[...] = q_acc[...] + q
        x_acc[...] = jnp.maximum(x_acc[...], x)
        n_acc[...] = jnp.minimum(n_acc[...], n)

    @pl.when(i == (N8 // LBLK8) - 1)
    def _():
        def _fold(v, op):
            parts = [v[:, k * LS:(k + 1) * LS] for k in range(8)]
            r = parts[0]
            for p in parts[1:]:
                r = op(r, p)
            return r

        s16 = _fold(s_acc[...], jnp.add)
        q16 = _fold(q_acc[...], jnp.add)
        x16 = _fold(x_acc[...], jnp.maximum)
        n16 = _fold(n_acc[...], jnp.minimum)
        inv = _f32(1.0 / N_LINKS)
        mean = s16 * inv
        var = q16 * inv - mean * mean
        std = jnp.sqrt(jnp.maximum(var, 0.0))
        ri = jnp.concatenate([mean, x16, n16, std], axis=1)
        r1 = jnp.tanh(
            jnp.dot(ri, w1_ref[...], preferred_element_type=_f32) + b1_ref[...])
        r2 = jnp.tanh(
            jnp.dot(r1, w2_ref[...], preferred_element_type=_f32) + b2_ref[...])
        o_ref[...] = (
            jnp.dot(r2, w3_ref[...], preferred_element_type=_f32) + b3_ref[...])


def _readout(ls, w1, b1, w2, b2, w3, b3):
    grid = (N8 // LBLK8,)
    return pl.pallas_call(
        _readout_body,
        grid=grid,
        in_specs=[
            pl.BlockSpec((LBLK8, 128), lambda i: (i, 0)),
            pl.BlockSpec((4 * LS, FIRST_H), lambda i: (0, 0)),
            pl.BlockSpec((1, FIRST_H), lambda i: (0, 0)),
            pl.BlockSpec((FIRST_H, FINAL_H), lambda i: (0, 0)),
            pl.BlockSpec((1, FINAL_H), lambda i: (0, 0)),
            pl.BlockSpec((FINAL_H, 1), lambda i: (0, 0)),
            pl.BlockSpec((1, 1), lambda i: (0, 0)),
        ],
        out_specs=pl.BlockSpec((1, 1), lambda i: (0, 0)),
        out_shape=jax.ShapeDtypeStruct((1, 1), _f32),
        scratch_shapes=[
            pltpu.VMEM((1, 128), _f32),
            pltpu.VMEM((1, 128), _f32),
            pltpu.VMEM((1, 128), _f32),
            pltpu.VMEM((1, 128), _f32),
        ],
    )(ls, w1, b1, w2, b2, w3, b3)


def _bd(w):
    return jnp.kron(jnp.eye(8, dtype=_f32), w)


def kernel(input, incoming_links, outcoming_links,
           cm_W1, cm_b1, cm_W2, cm_b2,
           lu_W1, lu_b1, lu_W2, lu_b2, lu_W3, lu_b3,
           ro_W1, ro_b1, ro_W2, ro_b2, ro_W3, ro_b3):
    i32 = jnp.int32
    ls0 = jnp.pad(jnp.transpose(input.reshape(NUM_FEATURES, N_LINKS)),
                  ((0, N_PAD - N_LINKS), (0, LS - NUM_FEATURES)))
    ls = ls0.reshape(N8, 128)

    dst = outcoming_links.astype(i32)
    inc = incoming_links.astype(i32)
    dst_s, inc_s = lax.sort_key_val(dst, inc)
    bnd = jnp.searchsorted(dst_s, jnp.arange(NW + 1, dtype=i32) * LPT,
                           ).astype(i32)
    bnd = jnp.pad(bnd, (0, 40 - (NW + 1)), constant_values=E)
    pad_e = jnp.zeros((E_PAD - E,), i32)
    dst_p = jnp.concatenate([dst_s, pad_e])
    inc_p = jnp.concatenate([inc_s, pad_e])

    cm_wa, cm_wb = _bd(cm_W1[:LS]), _bd(cm_W1[LS:])
    cm_w2 = _bd(cm_W2)
    lu_wa, lu_wb, lu_wc = (_bd(lu_W1[:LS]), _bd(lu_W1[LS:2 * LS]),
                           _bd(lu_W1[2 * LS:]))
    lu_w2, lu_w3 = _bd(lu_W2), _bd(lu_W3)
    cb1 = jnp.tile(cm_b1, 8).reshape(1, -1)
    cb2 = jnp.tile(cm_b2, 8).reshape(1, -1)
    lb1 = jnp.tile(lu_b1, 8).reshape(1, -1)
    lb2 = jnp.tile(lu_b2, 8).reshape(1, -1)
    lb3 = jnp.tile(lu_b3, 8).reshape(1, -1)
    rb1 = ro_b1.reshape(1, -1)
    rb2 = ro_b2.reshape(1, -1)
    rb3 = ro_b3.reshape(1, -1)

    inc2d = inc_p.reshape(E_PAD // GCH, GCH)
    dst2d = dst_p.reshape(E_PAD // GCH, GCH)

    for _ in range(ITERS):
        inc_g, outc_g = _sc_gather(ls.reshape(N_PAD, LS), inc2d, dst2d)
        msgs = _edge_mlp(inc_g.reshape(E8, 128), outc_g.reshape(E8, 128),
                         cm_wa, cm_wb, cb1, cm_w2, cb2)
        amax_p, amin_p = _sc_reduce(msgs.reshape(E_PAD, LS), dst_p, bnd)
        ls = _link_mlp(ls, amax_p.reshape(N8, 128), amin_p.reshape(N8, 128),
                       lu_wa, lu_wb, lu_wc, lb1, lu_w2, lb2, lu_w3, lb3)

    v = _readout(ls, ro_W1, rb1, ro_W2, rb2, ro_W3, rb3)
    return v.reshape(-1)
